# Initial kernel scaffold; baseline (speedup 1.0000x reference)
import jax
import jax.numpy as jnp
from jax.experimental import pallas as pl


def kernel(x_question, x_answer, pos_edge_label_index, neg_edge_label_index):
    raise NotImplementedError("write your pallas kernel here")


# speedup vs baseline: 4.7427x; 4.7427x over previous
import functools

import jax
import jax.numpy as jnp
from jax import lax
from jax.experimental import pallas as pl
from jax.experimental.pallas import tpu as pltpu
from jax.experimental.pallas import tpu_sc as plsc

NC = 2
NS = 16
NW = NC * NS
L = 16

CH = 64
B = 128
G = B // L


def _predict(n_edges):
  n_chunks = n_edges // B
  k_max = -(-n_chunks // NW)

  mesh = plsc.VectorSubcoreMesh(
      core_axis_name="c", subcore_axis_name="s", num_cores=NC,
      num_subcores=NS)

  @functools.partial(
      pl.kernel,
      out_type=(
          jax.ShapeDtypeStruct((n_edges,), jnp.float32),
          jax.ShapeDtypeStruct((n_edges,), jnp.float32),
      ),
      mesh=mesh,
      compiler_params=pltpu.CompilerParams(
          needs_layout_passes=False, use_tc_tiling_on_sc=False),
      scratch_types=dict(
          qidx_v=pltpu.VMEM((B,), jnp.int32),
          aidx_v=pltpu.VMEM((B,), jnp.int32),
          rows_q=pltpu.VMEM((B, CH), jnp.float32),
          rows_a=pltpu.VMEM((B, CH), jnp.float32),
          partials=pltpu.VMEM((B * L,), jnp.float32),
          out_v=pltpu.VMEM((B,), jnp.float32),
          sem_q=pltpu.SemaphoreType.DMA,
          sem_a=pltpu.SemaphoreType.DMA,
      ),
  )
  def sc_kernel(xq_hbm, xa_hbm, qp_hbm, ap_hbm, qn_hbm, an_hbm,
                pos_hbm, neg_hbm, *, qidx_v, aidx_v, rows_q, rows_a,
                partials, out_v, sem_q, sem_a):
    wid = lax.axis_index("s") * NC + lax.axis_index("c")
    lane = lax.iota(jnp.int32, L)

    def do_chunk(q_hbm, a_hbm, out_hbm, cid):
      base = cid * B
      pltpu.sync_copy(q_hbm.at[pl.ds(base, B)], qidx_v)
      pltpu.sync_copy(a_hbm.at[pl.ds(base, B)], aidx_v)
      cq = pltpu.async_copy(xq_hbm.at[qidx_v], rows_q, sem_q)
      ca = pltpu.async_copy(xa_hbm.at[aidx_v], rows_a, sem_a)
      cq.wait()
      ca.wait()

      def edge_body(e, carry):
        p = rows_q[e, pl.ds(0, L)] * rows_a[e, pl.ds(0, L)]
        for k in range(1, CH // L):
          p = p + rows_q[e, pl.ds(k * L, L)] * rows_a[e, pl.ds(k * L, L)]
        partials[pl.ds(e * L, L)] = p
        return carry

      lax.fori_loop(0, B, edge_body, 0)

      def group_body(g, carry):
        base = g * (L * L) + lane * L
        acc = plsc.load_gather(partials, [base])
        for l in range(1, L):
          acc = acc + plsc.load_gather(partials, [base + l])
        out_v[pl.ds(g * L, L)] = acc
        return carry

      lax.fori_loop(0, G, group_body, 0)
      pltpu.sync_copy(out_v, out_hbm.at[pl.ds(base, B)])

    def chunk_loop(k, carry):
      cid = wid + k * NW

      @pl.when(cid < n_chunks)
      def _():
        do_chunk(qp_hbm, ap_hbm, pos_hbm, cid)
        do_chunk(qn_hbm, an_hbm, neg_hbm, cid)

      return carry

    lax.fori_loop(0, k_max, chunk_loop, 0)

  return sc_kernel


def kernel(x_question, x_answer, pos_edge_label_index, neg_edge_label_index):
  n_edges = pos_edge_label_index.shape[1]
  qp = pos_edge_label_index[0].astype(jnp.int32)
  ap = pos_edge_label_index[1].astype(jnp.int32)
  qn = neg_edge_label_index[0].astype(jnp.int32)
  an = neg_edge_label_index[1].astype(jnp.int32)
  return _predict(n_edges)(x_question, x_answer, qp, ap, qn, an)


# speedup vs baseline: 9.3311x; 1.9675x over previous
import functools

import jax
import jax.numpy as jnp
from jax import lax
from jax.experimental import pallas as pl
from jax.experimental.pallas import tpu as pltpu
from jax.experimental.pallas import tpu_sc as plsc

NC = 2
NS = 16
NW = NC * NS
L = 16

CH = 64
B = 128
SUBS = 16
SE = B * SUBS


def _predict(n_total):
  assert n_total % NW == 0
  epw = n_total // NW
  n_super = epw // SE
  tail = epw - n_super * SE
  tail_full = tail // B
  tail_rem = tail % B
  assert n_super % 2 == 0 and tail_rem % L == 0
  assert tail_full < SUBS

  mesh = plsc.VectorSubcoreMesh(
      core_axis_name="c", subcore_axis_name="s", num_cores=NC,
      num_subcores=NS)

  @functools.partial(
      pl.kernel,
      out_type=jax.ShapeDtypeStruct((n_total,), jnp.float32),
      mesh=mesh,
      compiler_params=pltpu.CompilerParams(
          needs_layout_passes=False, use_tc_tiling_on_sc=False),
      scratch_types=dict(
          qidx0=pltpu.VMEM((SE,), jnp.int32),
          qidx1=pltpu.VMEM((SE,), jnp.int32),
          aidx0=pltpu.VMEM((SE,), jnp.int32),
          aidx1=pltpu.VMEM((SE,), jnp.int32),
          rq0=pltpu.VMEM((B, CH), jnp.float32),
          rq1=pltpu.VMEM((B, CH), jnp.float32),
          ra0=pltpu.VMEM((B, CH), jnp.float32),
          ra1=pltpu.VMEM((B, CH), jnp.float32),
          out0=pltpu.VMEM((SE,), jnp.float32),
          out1=pltpu.VMEM((SE,), jnp.float32),
          partials=pltpu.VMEM((B * L,), jnp.float32),
          siq0=pltpu.SemaphoreType.DMA,
          siq1=pltpu.SemaphoreType.DMA,
          sia0=pltpu.SemaphoreType.DMA,
          sia1=pltpu.SemaphoreType.DMA,
          sgq0=pltpu.SemaphoreType.DMA,
          sgq1=pltpu.SemaphoreType.DMA,
          sga0=pltpu.SemaphoreType.DMA,
          sga1=pltpu.SemaphoreType.DMA,
          so0=pltpu.SemaphoreType.DMA,
          so1=pltpu.SemaphoreType.DMA,
      ),
  )
  def sc_kernel(xq_hbm, xa_hbm, qi_hbm, ai_hbm, out_hbm, *,
                qidx0, qidx1, aidx0, aidx1, rq0, rq1, ra0, ra1, out0, out1,
                partials, siq0, siq1, sia0, sia1, sgq0, sgq1, sga0, sga1,
                so0, so1):
    wid = lax.axis_index("s") * NC + lax.axis_index("c")
    wbase = wid * epw
    lane = lax.iota(jnp.int32, L)
    ibufs = ((qidx0, aidx0, siq0, sia0), (qidx1, aidx1, siq1, sia1))
    rbufs = ((rq0, ra0, sgq0, sga0), (rq1, ra1, sgq1, sga1))
    obufs = ((out0, so0), (out1, so1))

    def idx_start(base, n, ib):
      qb, ab, sq, sa = ib
      pltpu.async_copy(qi_hbm.at[pl.ds(base, n)], qb.at[pl.ds(0, n)], sq)
      pltpu.async_copy(ai_hbm.at[pl.ds(base, n)], ab.at[pl.ds(0, n)], sa)

    def idx_wait(n, ib):
      qb, ab, sq, sa = ib
      pltpu.make_async_copy(
          qi_hbm.at[pl.ds(0, n)], qb.at[pl.ds(0, n)], sq).wait()
      pltpu.make_async_copy(
          ai_hbm.at[pl.ds(0, n)], ab.at[pl.ds(0, n)], sa).wait()

    def gather_start(ib, off, n, rb):
      qb, ab, _, _ = ib
      rq, ra, sgq, sga = rb
      pltpu.async_copy(
          xq_hbm.at[qb.at[pl.ds(off, n)]], rq.at[pl.ds(0, n)], sgq)
      pltpu.async_copy(
          xa_hbm.at[ab.at[pl.ds(off, n)]], ra.at[pl.ds(0, n)], sga)

    def gather_wait(ib, n, rb):
      qb, ab, _, _ = ib
      rq, ra, sgq, sga = rb
      pltpu.make_async_copy(
          xq_hbm.at[qb.at[pl.ds(0, n)]], rq.at[pl.ds(0, n)], sgq).wait()
      pltpu.make_async_copy(
          xa_hbm.at[ab.at[pl.ds(0, n)]], ra.at[pl.ds(0, n)], sga).wait()

    def compute(rb, ob, out_off, ngroups):
      rq, ra, _, _ = rb
      outbuf, _ = ob
      n = ngroups * L

      def edge_body(e, carry):
        p = rq[e, pl.ds(0, L)] * ra[e, pl.ds(0, L)]
        for k in range(1, CH // L):
          p = p + rq[e, pl.ds(k * L, L)] * ra[e, pl.ds(k * L, L)]
        partials[pl.ds(e * L, L)] = p
        return carry

      lax.fori_loop(0, n, edge_body, 0)

      def group_body(g, carry):
        base = g * (L * L) + lane * L
        acc = plsc.load_gather(partials, [base])
        for l in range(1, L):
          acc = acc + plsc.load_gather(partials, [base + l])
        outbuf[pl.ds(out_off + g * L, L)] = acc
        return carry

      lax.fori_loop(0, ngroups, group_body, 0)

    def out_start(base, n, ob):
      outbuf, so = ob
      pltpu.async_copy(outbuf.at[pl.ds(0, n)], out_hbm.at[pl.ds(base, n)], so)

    def out_wait(n, ob):
      outbuf, so = ob
      pltpu.make_async_copy(
          outbuf.at[pl.ds(0, n)], out_hbm.at[pl.ds(0, n)], so).wait()

    idx_start(wbase, SE, ibufs[0])
    if n_super > 1:
      idx_start(wbase + SE, SE, ibufs[1])

    def super_body(s, ibi):
      ib = ibufs[ibi]
      ob = obufs[ibi]
      sbase = wbase + s * SE
      idx_wait(SE, ib)

      @pl.when(s >= 2)
      def _():
        out_wait(SE, ob)

      gather_start(ib, 0, B, rbufs[0])
      gather_start(ib, B, B, rbufs[1])

      def jj_body(jj, carry):
        for par in range(2):
          j = 2 * jj + par
          gather_wait(ib, B, rbufs[par])
          compute(rbufs[par], ob, j * B, B // L)

          @pl.when(j + 2 < SUBS)
          def _():
            gather_start(ib, (j + 2) * B, B, rbufs[par])

        return carry

      lax.fori_loop(0, SUBS // 2, jj_body, 0)
      out_start(sbase, SE, ob)

      nxt = s + 2

      @pl.when(nxt < n_super)
      def _():
        idx_start(wbase + nxt * SE, SE, ib)

      if tail:
        @pl.when(nxt == n_super)
        def _():
          idx_start(wbase + n_super * SE, tail, ib)

    def pair_body(s2, carry):
      super_body(2 * s2, 0)
      super_body(2 * s2 + 1, 1)
      return carry

    lax.fori_loop(0, n_super // 2, pair_body, 0)

    if tail:
      ib = ibufs[0]
      ob = obufs[0]
      tbase = wbase + n_super * SE
      idx_wait(tail, ib)
      if n_super >= 2:
        out_wait(SE, ob)
      n_subs = tail_full + (1 if tail_rem else 0)
      sizes = [B] * tail_full + ([tail_rem] if tail_rem else [])
      for j in range(min(2, n_subs)):
        gather_start(ib, j * B, sizes[j], rbufs[j % 2])
      for j in range(n_subs):
        gather_wait(ib, sizes[j], rbufs[j % 2])
        compute(rbufs[j % 2], ob, j * B, sizes[j] // L)
        if j + 2 < n_subs:
          gather_start(ib, (j + 2) * B, sizes[j + 2], rbufs[j % 2])
      out_start(tbase, tail, ob)

    if n_super >= 1:
      out_wait(SE, obufs[1] if n_super % 2 == 0 else obufs[0])
    if tail:
      out_wait(tail, obufs[0])
    elif n_super >= 2:
      out_wait(SE, obufs[0] if n_super % 2 == 0 else obufs[1])

  return sc_kernel


def kernel(x_question, x_answer, pos_edge_label_index, neg_edge_label_index):
  n_edges = pos_edge_label_index.shape[1]
  qi = jnp.concatenate([pos_edge_label_index[0], neg_edge_label_index[0]])
  ai = jnp.concatenate([pos_edge_label_index[1], neg_edge_label_index[1]])
  pred = _predict(2 * n_edges)(
      x_question, x_answer, qi.astype(jnp.int32), ai.astype(jnp.int32))
  return (pred[:n_edges], pred[n_edges:])
